# Initial kernel scaffold; baseline (speedup 1.0000x reference)
import jax
import jax.numpy as jnp
from jax.experimental import pallas as pl


def kernel(x, edge_index, Wl1, Wr1, att1, b1, Wl2, Wr2, att2, b2):
    raise NotImplementedError("write your pallas kernel here")


# speedup vs baseline: 273.1926x; 273.1926x over previous
import functools

import jax
import jax.numpy as jnp
from jax import lax
from jax.experimental import pallas as pl
from jax.experimental.pallas import tpu as pltpu
from jax.experimental.pallas import tpu_sc as plsc

f32 = jnp.float32
i32 = jnp.int32

NC = 2
NS = 16
L = 16
NW = NC * NS
BLK = 128
ZR = 64


def _fori(hi, body):
  return lax.fori_loop(jnp.int32(0), jnp.int32(hi), body, jnp.int32(0))


def _compute_l1(xjs, xis, atts, sw_ref):
  ts = []
  for v in range(4):
    z = xis[v] + xjs[v]
    lr = jnp.maximum(z, 0.2 * z)
    ts.append(lr * atts[v])
  u = (ts[0] + ts[1]) + (ts[2] + ts[3])
  sw_ref[pl.ds(0, L)] = u
  sw_ref[pl.ds(L, L)] = u
  u = u + sw_ref[pl.ds(8, L)]
  a = jnp.exp(u)
  return [xjs[v] * a for v in range(4)], a


def _compute_l2(xjs, xis, atts, sw_ref):
  del sw_ref
  z = xis[0] + xjs[0]
  lr = jnp.maximum(z, 0.2 * z)
  t = lr * atts[0]
  u = jnp.broadcast_to(jnp.sum(t), (L,))
  a = jnp.exp(u)
  return [xjs[0] * a], a


def _make_edge_kernel(n_tbl, F, nblk, E, Et, compute_fn, NR):
  per_tile = nblk * BLK
  STRIDE = NR // NS
  TRASH = NR - 8
  NV = F // L
  mesh = plsc.VectorSubcoreMesh(
      core_axis_name="c", subcore_axis_name="s",
      num_cores=NC, num_subcores=NS)

  @functools.partial(
      pl.kernel,
      out_type=(jax.ShapeDtypeStruct((NC, NR, F), f32),
                jax.ShapeDtypeStruct((NC, NR, L), f32)),
      mesh=mesh,
      compiler_params=pltpu.CompilerParams(use_tc_tiling_on_sc=False, needs_layout_passes=False),
      scratch_types=[
          pltpu.VMEM((BLK,), i32),
          pltpu.VMEM((BLK,), i32),
          pltpu.VMEM((BLK,), i32),
          pltpu.VMEM((BLK, F), f32),
          pltpu.VMEM((BLK, F), f32),
          pltpu.VMEM((BLK, F), f32),
          pltpu.VMEM((BLK, L), f32),
          pltpu.VMEM((F,), f32),
          pltpu.VMEM((2 * L,), f32),
          pltpu.VMEM((ZR, F), f32),
          pltpu.VMEM((ZR, L), f32),
          pltpu.VMEM_SHARED((NR, F), f32),
          pltpu.VMEM_SHARED((NR, L), f32),
          pltpu.SemaphoreType.DMA,
          pltpu.SemaphoreType.DMA,
      ],
  )
  def k(tbl_l, tbl_r, srcp, dstp, attp, outm, outs,
        src_v, dst_v, scat_v, xj_v, xi_v, msg_v, alp_v, att_v, sw_v,
        zm_v, zs_v, accm_sh, accs_sh, sem1, sem2):
    cid = lax.axis_index("c")
    sid = lax.axis_index("s")
    wid = cid * NS + sid

    zero16 = jnp.zeros((L,), f32)

    def zrow(r, carry):
      for v in range(NV):
        zm_v[r, pl.ds(v * L, L)] = zero16
      zs_v[r, pl.ds(0, L)] = zero16
      return carry
    _fori(ZR, zrow)

    def zcopy(kk, carry):
      off = sid * STRIDE + kk * ZR
      pltpu.sync_copy(zm_v, accm_sh.at[pl.ds(off, ZR)])
      pltpu.sync_copy(zs_v, accs_sh.at[pl.ds(off, ZR)])
      return carry
    _fori(STRIDE // ZR, zcopy)
    plsc.subcore_barrier()

    pltpu.sync_copy(attp, att_v)
    atts = [att_v[pl.ds(v * L, L)] for v in range(NV)]

    tile_base = wid * per_tile

    def blk_body(b, carry):
      base = tile_base + b * BLK
      pltpu.sync_copy(srcp.at[pl.ds(base, BLK)], src_v)
      pltpu.sync_copy(dstp.at[pl.ds(base, BLK)], dst_v)
      cp1 = pltpu.async_copy(tbl_l.at[src_v], xj_v, sem1)
      cp2 = pltpu.async_copy(tbl_r.at[dst_v], xi_v, sem2)
      for g in range(BLK // L):
        sv = src_v[pl.ds(g * L, L)]
        dv = dst_v[pl.ds(g * L, L)]
        eid = base + g * L + lax.iota(i32, L)
        valid = jnp.where(eid < E, sv != dv, eid < Et)
        scat_v[pl.ds(g * L, L)] = jnp.where(valid, dv, TRASH)
      cp1.wait()
      cp2.wait()

      def edge(e, ecarry):
        xjs = [xj_v[e, pl.ds(v * L, L)] for v in range(NV)]
        xis = [xi_v[e, pl.ds(v * L, L)] for v in range(NV)]
        msgs, a = compute_fn(xjs, xis, atts, sw_v)
        for v in range(NV):
          msg_v[e, pl.ds(v * L, L)] = msgs[v]
        alp_v[e, pl.ds(0, L)] = a
        return ecarry
      _fori(BLK, edge)

      pltpu.sync_copy(msg_v, accm_sh.at[scat_v], add=True)
      pltpu.sync_copy(alp_v, accs_sh.at[scat_v], add=True)
      return carry
    _fori(nblk, blk_body)

    plsc.subcore_barrier()
    off = sid * STRIDE
    pltpu.sync_copy(accm_sh.at[pl.ds(off, STRIDE)],
                    outm.at[cid, pl.ds(off, STRIDE)])
    pltpu.sync_copy(accs_sh.at[pl.ds(off, STRIDE)],
                    outs.at[cid, pl.ds(off, STRIDE)])

  return k


def _tc_linear2(x, wl, wr):
  n = x.shape[0]
  m = wl.shape[1]

  def body(x_ref, wl_ref, wr_ref, ol_ref, or_ref):
    xv = x_ref[...]
    ol_ref[...] = jnp.dot(xv, wl_ref[...], preferred_element_type=f32,
                         precision=lax.Precision.HIGHEST)
    or_ref[...] = jnp.dot(xv, wr_ref[...], preferred_element_type=f32,
                         precision=lax.Precision.HIGHEST)

  return pl.pallas_call(
      body,
      out_shape=(jax.ShapeDtypeStruct((n, m), f32),
                 jax.ShapeDtypeStruct((n, m), f32)),
  )(x, wl, wr)


def _tc_combine1(pm, ps, b1p, wl2, wr2, ch):
  NR = pm.shape[1]
  F = pm.shape[2]
  m = wl2.shape[1]

  def body(pm_ref, ps_ref, b_ref, wl_ref, wr_ref, ol_ref, or_ref):
    accm = pm_ref[0] + pm_ref[1]
    accs = ps_ref[0] + ps_ref[1]
    denom = jnp.tile(accs, (1, F // L)) + 1e-16
    h = accm / denom + b_ref[...]
    h = jnp.where(h > 0, h, jnp.exp(h) - 1.0)
    ol_ref[...] = jnp.dot(h, wl_ref[...], preferred_element_type=f32,
                          precision=lax.Precision.HIGHEST)
    or_ref[...] = jnp.dot(h, wr_ref[...], preferred_element_type=f32,
                          precision=lax.Precision.HIGHEST)

  return pl.pallas_call(
      body,
      out_shape=(jax.ShapeDtypeStruct((NR, m), f32),
                 jax.ShapeDtypeStruct((NR, m), f32)),
  )(pm, ps, b1p, wl2, wr2)


def _tc_finalize(pm, ps, b2, n, dout):
  def body(pm_ref, ps_ref, b_ref, h2_ref, out_ref, pred_ref):
    accm = pm_ref[0] + pm_ref[1]
    accs = ps_ref[0] + ps_ref[1]
    h2full = accm / (accs + 1e-16) + b_ref[...]
    h2 = h2full[:n]
    mx = jnp.max(h2, axis=1, keepdims=True)
    lse = jnp.log(jnp.sum(jnp.exp(h2 - mx), axis=1, keepdims=True)) + mx
    o = h2 - lse
    h2_ref[...] = h2
    out_ref[...] = o
    eq = o >= jnp.max(o, axis=1, keepdims=True)
    idx = lax.broadcasted_iota(i32, o.shape, 1)
    pred_ref[...] = jnp.min(jnp.where(eq, idx, dout), axis=1, keepdims=True)

  return pl.pallas_call(
      body,
      out_shape=(jax.ShapeDtypeStruct((n, dout), f32),
                 jax.ShapeDtypeStruct((n, dout), f32),
                 jax.ShapeDtypeStruct((n, 1), i32)),
  )(pm, ps, b2)


def kernel(x, edge_index, Wl1, Wr1, att1, b1, Wl2, Wr2, att2, b2):
  N, DIN = x.shape
  E = edge_index.shape[1]
  H, C = att1.shape
  HC = H * C
  DOUT = Wl2.shape[1]

  NR = -(-N // (NS * ZR)) * (NS * ZR)

  x32 = x.astype(f32)
  Wl1p = Wl1.astype(f32).reshape(DIN, H, C).transpose(0, 2, 1).reshape(DIN, HC)
  Wr1p = Wr1.astype(f32).reshape(DIN, H, C).transpose(0, 2, 1).reshape(DIN, HC)
  att1p = att1.astype(f32).T.reshape(HC)
  b1p = b1.astype(f32).reshape(H, C).T.reshape(1, HC)
  Wl2p = Wl2.astype(f32).reshape(H, C, DOUT).transpose(1, 0, 2).reshape(HC, DOUT)
  Wr2p = Wr2.astype(f32).reshape(H, C, DOUT).transpose(1, 0, 2).reshape(HC, DOUT)
  att2p = att2.astype(f32).reshape(DOUT)
  b2p = b2.astype(f32).reshape(1, DOUT)

  src = edge_index[0].astype(i32)
  dst = edge_index[1].astype(i32)
  loop = jnp.arange(N, dtype=i32)
  Et = E + N
  nblk = -(-Et // (NW * BLK))
  Etp = nblk * BLK * NW
  pad = Etp - Et
  srcp = jnp.concatenate([src, loop, jnp.zeros((pad,), i32)])
  dstp = jnp.concatenate([dst, loop, jnp.zeros((pad,), i32)])

  xl1, xr1 = _tc_linear2(x32, Wl1p, Wr1p)
  edge1 = _make_edge_kernel(N, HC, nblk, E, Et, _compute_l1, NR)
  pm1, ps1 = edge1(xl1, xr1, srcp, dstp, att1p)

  xl2, xr2 = _tc_combine1(pm1, ps1, b1p, Wl2p, Wr2p, C)

  edge2 = _make_edge_kernel(NR, DOUT, nblk, E, Et, _compute_l2, NR)
  pm2, ps2 = edge2(xl2, xr2, srcp, dstp, att2p)

  h2, out, preds = _tc_finalize(pm2, ps2, b2p, N, DOUT)
  return (h2.astype(jnp.float64), out.astype(jnp.float64),
          preds.reshape(N).astype(jnp.int64))


# speedup vs baseline: 335.8612x; 1.2294x over previous
import functools

import jax
import jax.numpy as jnp
from jax import lax
from jax.experimental import pallas as pl
from jax.experimental.pallas import tpu as pltpu
from jax.experimental.pallas import tpu_sc as plsc

f32 = jnp.float32
i32 = jnp.int32

NC = 2
NS = 16
L = 16
NW = NC * NS
BLK = 128
ZR = 64


def _fori(hi, body):
  return lax.fori_loop(jnp.int32(0), jnp.int32(hi), body, jnp.int32(0))


def _compute_l1(xjs, xis, atts, sw_ref):
  ts = []
  for v in range(4):
    z = xis[v] + xjs[v]
    lr = jnp.maximum(z, 0.2 * z)
    ts.append(lr * atts[v])
  u = (ts[0] + ts[1]) + (ts[2] + ts[3])
  sw_ref[pl.ds(0, L)] = u
  sw_ref[pl.ds(L, L)] = u
  u = u + sw_ref[pl.ds(8, L)]
  a = jnp.exp(u)
  return [xjs[v] * a for v in range(4)], a


def _compute_l2(xjs, xis, atts, sw_ref):
  del sw_ref
  z = xis[0] + xjs[0]
  lr = jnp.maximum(z, 0.2 * z)
  t = lr * atts[0]
  u = jnp.broadcast_to(jnp.sum(t), (L,))
  a = jnp.exp(u)
  return [xjs[0] * a], a


def _make_edge_kernel(n_tbl, F, nblk, E, Et, compute_fn, NR):
  assert nblk % 2 == 0
  per_tile = nblk * BLK
  STRIDE = NR // NS
  TRASH = NR - 8
  NV = F // L
  U = 4
  mesh = plsc.VectorSubcoreMesh(
      core_axis_name="c", subcore_axis_name="s",
      num_cores=NC, num_subcores=NS)

  @functools.partial(
      pl.kernel,
      out_type=(jax.ShapeDtypeStruct((NC, NR, F), f32),
                jax.ShapeDtypeStruct((NC, NR, L), f32)),
      mesh=mesh,
      compiler_params=pltpu.CompilerParams(use_tc_tiling_on_sc=False,
                                           needs_layout_passes=False),
      scratch_types=[
          pltpu.VMEM((BLK,), i32),
          pltpu.VMEM((BLK,), i32),
          pltpu.VMEM((BLK, F), f32),
          pltpu.VMEM((BLK, F), f32),
          pltpu.VMEM((BLK,), i32),
          pltpu.VMEM((BLK,), i32),
          pltpu.VMEM((BLK, F), f32),
          pltpu.VMEM((BLK, F), f32),
          pltpu.VMEM((BLK,), i32),
          pltpu.VMEM((BLK, F), f32),
          pltpu.VMEM((BLK, L), f32),
          pltpu.VMEM((F,), f32),
          pltpu.VMEM((2 * L,), f32),
          pltpu.VMEM((2 * L,), f32),
          pltpu.VMEM((2 * L,), f32),
          pltpu.VMEM((2 * L,), f32),
          pltpu.VMEM((ZR, F), f32),
          pltpu.VMEM((ZR, L), f32),
          pltpu.VMEM_SHARED((NR, F), f32),
          pltpu.VMEM_SHARED((NR, L), f32),
          pltpu.SemaphoreType.DMA,
          pltpu.SemaphoreType.DMA,
      ],
  )
  def k(tbl_l, tbl_r, srcp, dstp, attp, outm, outs,
        srcA, dstA, xjA, xiA, srcB, dstB, xjB, xiB,
        scat_v, msg_v, alp_v, att_v, sw0, sw1, sw2, sw3,
        zm_v, zs_v, accm_sh, accs_sh, semA, semB):
    cid = lax.axis_index("c")
    sid = lax.axis_index("s")
    wid = cid * NS + sid
    sws = [sw0, sw1, sw2, sw3]
    bufA = (srcA, dstA, xjA, xiA, semA)
    bufB = (srcB, dstB, xjB, xiB, semB)

    zero16 = jnp.zeros((L,), f32)

    def zrow(r, carry):
      for v in range(NV):
        zm_v[r, pl.ds(v * L, L)] = zero16
      zs_v[r, pl.ds(0, L)] = zero16
      return carry
    _fori(ZR, zrow)

    def zcopy(kk, carry):
      off = sid * STRIDE + kk * ZR
      pltpu.sync_copy(zm_v, accm_sh.at[pl.ds(off, ZR)])
      pltpu.sync_copy(zs_v, accs_sh.at[pl.ds(off, ZR)])
      return carry
    _fori(STRIDE // ZR, zcopy)
    plsc.subcore_barrier()

    pltpu.sync_copy(attp, att_v)
    atts = [att_v[pl.ds(v * L, L)] for v in range(NV)]

    tile_base = wid * per_tile

    def start_block(b, buf):
      src_v, dst_v, xj_v, xi_v, sem = buf
      base = tile_base + b * BLK
      pltpu.sync_copy(srcp.at[pl.ds(base, BLK)], src_v)
      pltpu.sync_copy(dstp.at[pl.ds(base, BLK)], dst_v)
      pltpu.async_copy(tbl_l.at[src_v], xj_v, sem)
      pltpu.async_copy(tbl_r.at[dst_v], xi_v, sem)

    def process_block(b, cur, nxt):
      src_v, dst_v, xj_v, xi_v, sem = cur

      @pl.when(b + 1 < nblk)
      def _():
        start_block(b + 1, nxt)

      base = tile_base + b * BLK
      for g in range(BLK // L):
        sv = src_v[pl.ds(g * L, L)]
        dv = dst_v[pl.ds(g * L, L)]
        eid = base + g * L + lax.iota(i32, L)
        valid = jnp.where(eid < E, sv != dv, eid < Et)
        scat_v[pl.ds(g * L, L)] = jnp.where(valid, dv, TRASH)

      pltpu.make_async_copy(tbl_l.at[pl.ds(0, BLK)], xj_v, sem).wait()
      pltpu.make_async_copy(tbl_l.at[pl.ds(0, BLK)], xi_v, sem).wait()

      def edge(ii, ecarry):
        e0 = ii * U
        for u in range(U):
          e = e0 + u
          xjs = [xj_v[e, pl.ds(v * L, L)] for v in range(NV)]
          xis = [xi_v[e, pl.ds(v * L, L)] for v in range(NV)]
          msgs, a = compute_fn(xjs, xis, atts, sws[u])
          for v in range(NV):
            msg_v[e, pl.ds(v * L, L)] = msgs[v]
          alp_v[e, pl.ds(0, L)] = a
        return ecarry
      _fori(BLK // U, edge)

      pltpu.sync_copy(msg_v, accm_sh.at[scat_v], add=True)
      pltpu.sync_copy(alp_v, accs_sh.at[scat_v], add=True)

    start_block(jnp.int32(0), bufA)

    def pair_body(i, carry):
      b = i * 2
      process_block(b, bufA, bufB)
      process_block(b + 1, bufB, bufA)
      return carry
    _fori(nblk // 2, pair_body)

    plsc.subcore_barrier()
    off = sid * STRIDE
    pltpu.sync_copy(accm_sh.at[pl.ds(off, STRIDE)],
                    outm.at[cid, pl.ds(off, STRIDE)])
    pltpu.sync_copy(accs_sh.at[pl.ds(off, STRIDE)],
                    outs.at[cid, pl.ds(off, STRIDE)])

  return k


def _tc_linear2(x, wl, wr):
  n = x.shape[0]
  m = wl.shape[1]

  def body(x_ref, wl_ref, wr_ref, ol_ref, or_ref):
    xv = x_ref[...]
    ol_ref[...] = jnp.dot(xv, wl_ref[...], preferred_element_type=f32,
                         precision=lax.Precision.HIGHEST)
    or_ref[...] = jnp.dot(xv, wr_ref[...], preferred_element_type=f32,
                         precision=lax.Precision.HIGHEST)

  return pl.pallas_call(
      body,
      out_shape=(jax.ShapeDtypeStruct((n, m), f32),
                 jax.ShapeDtypeStruct((n, m), f32)),
  )(x, wl, wr)


def _tc_combine1(pm, ps, b1p, wl2, wr2, ch):
  NR = pm.shape[1]
  F = pm.shape[2]
  m = wl2.shape[1]

  def body(pm_ref, ps_ref, b_ref, wl_ref, wr_ref, ol_ref, or_ref):
    accm = pm_ref[0] + pm_ref[1]
    accs = ps_ref[0] + ps_ref[1]
    denom = jnp.tile(accs, (1, F // L)) + 1e-16
    h = accm / denom + b_ref[...]
    h = jnp.where(h > 0, h, jnp.exp(h) - 1.0)
    ol_ref[...] = jnp.dot(h, wl_ref[...], preferred_element_type=f32,
                          precision=lax.Precision.HIGHEST)
    or_ref[...] = jnp.dot(h, wr_ref[...], preferred_element_type=f32,
                          precision=lax.Precision.HIGHEST)

  return pl.pallas_call(
      body,
      out_shape=(jax.ShapeDtypeStruct((NR, m), f32),
                 jax.ShapeDtypeStruct((NR, m), f32)),
  )(pm, ps, b1p, wl2, wr2)


def _tc_finalize(pm, ps, b2, n, dout):
  def body(pm_ref, ps_ref, b_ref, h2_ref, out_ref, pred_ref):
    accm = pm_ref[0] + pm_ref[1]
    accs = ps_ref[0] + ps_ref[1]
    h2full = accm / (accs + 1e-16) + b_ref[...]
    h2 = h2full[:n]
    mx = jnp.max(h2, axis=1, keepdims=True)
    lse = jnp.log(jnp.sum(jnp.exp(h2 - mx), axis=1, keepdims=True)) + mx
    o = h2 - lse
    h2_ref[...] = h2
    out_ref[...] = o
    eq = o >= jnp.max(o, axis=1, keepdims=True)
    idx = lax.broadcasted_iota(i32, o.shape, 1)
    pred_ref[...] = jnp.min(jnp.where(eq, idx, dout), axis=1, keepdims=True)

  return pl.pallas_call(
      body,
      out_shape=(jax.ShapeDtypeStruct((n, dout), f32),
                 jax.ShapeDtypeStruct((n, dout), f32),
                 jax.ShapeDtypeStruct((n, 1), i32)),
  )(pm, ps, b2)


def kernel(x, edge_index, Wl1, Wr1, att1, b1, Wl2, Wr2, att2, b2):
  N, DIN = x.shape
  E = edge_index.shape[1]
  H, C = att1.shape
  HC = H * C
  DOUT = Wl2.shape[1]

  NR = -(-N // (NS * ZR)) * (NS * ZR)

  x32 = x.astype(f32)
  Wl1p = Wl1.astype(f32).reshape(DIN, H, C).transpose(0, 2, 1).reshape(DIN, HC)
  Wr1p = Wr1.astype(f32).reshape(DIN, H, C).transpose(0, 2, 1).reshape(DIN, HC)
  att1p = att1.astype(f32).T.reshape(HC)
  b1p = b1.astype(f32).reshape(H, C).T.reshape(1, HC)
  Wl2p = Wl2.astype(f32).reshape(H, C, DOUT).transpose(1, 0, 2).reshape(HC, DOUT)
  Wr2p = Wr2.astype(f32).reshape(H, C, DOUT).transpose(1, 0, 2).reshape(HC, DOUT)
  att2p = att2.astype(f32).reshape(DOUT)
  b2p = b2.astype(f32).reshape(1, DOUT)

  src = edge_index[0].astype(i32)
  dst = edge_index[1].astype(i32)
  loop = jnp.arange(N, dtype=i32)
  Et = E + N
  nblk = 2 * (-(-Et // (NW * BLK * 2)))
  Etp = nblk * BLK * NW
  pad = Etp - Et
  srcp = jnp.concatenate([src, loop, jnp.zeros((pad,), i32)])
  dstp = jnp.concatenate([dst, loop, jnp.zeros((pad,), i32)])

  xl1, xr1 = _tc_linear2(x32, Wl1p, Wr1p)
  edge1 = _make_edge_kernel(N, HC, nblk, E, Et, _compute_l1, NR)
  pm1, ps1 = edge1(xl1, xr1, srcp, dstp, att1p)

  xl2, xr2 = _tc_combine1(pm1, ps1, b1p, Wl2p, Wr2p, C)

  edge2 = _make_edge_kernel(NR, DOUT, nblk, E, Et, _compute_l2, NR)
  pm2, ps2 = edge2(xl2, xr2, srcp, dstp, att2p)

  h2, out, preds = _tc_finalize(pm2, ps2, b2p, N, DOUT)
  return (h2.astype(jnp.float64), out.astype(jnp.float64),
          preds.reshape(N).astype(jnp.int64))


# speedup vs baseline: 362.1662x; 1.0783x over previous
import functools

import jax
import jax.numpy as jnp
from jax import lax
from jax.experimental import pallas as pl
from jax.experimental.pallas import tpu as pltpu
from jax.experimental.pallas import tpu_sc as plsc

f32 = jnp.float32
i32 = jnp.int32

NC = 2
NS = 16
L = 16
NW = NC * NS
BLK = 128
ZR = 64


def _fori(hi, body):
  return lax.fori_loop(jnp.int32(0), jnp.int32(hi), body, jnp.int32(0))


def _compute_l1(xjs, xis, atts, sw_ref):
  ts = []
  for v in range(4):
    z = xis[v] + xjs[v]
    lr = jnp.maximum(z, 0.2 * z)
    ts.append(lr * atts[v])
  u = (ts[0] + ts[1]) + (ts[2] + ts[3])
  sw_ref[pl.ds(0, L)] = u
  sw_ref[pl.ds(L, L)] = u
  u = u + sw_ref[pl.ds(8, L)]
  a = jnp.exp(u)
  return [xjs[v] * a for v in range(4)], a


def _compute_l2(xjs, xis, atts, sw_ref):
  del sw_ref
  z = xis[0] + xjs[0]
  lr = jnp.maximum(z, 0.2 * z)
  t = lr * atts[0]
  u = jnp.broadcast_to(jnp.sum(t), (L,))
  a = jnp.exp(u)
  return [xjs[0] * a], a


def _make_edge_kernel(n_tbl, F, nblk, E, Et, compute_fn, NR):
  assert nblk % 2 == 0
  per_tile = nblk * BLK
  STRIDE = NR // NS
  TRASH = NR - 8
  NV = F // L
  U = 4
  mesh = plsc.VectorSubcoreMesh(
      core_axis_name="c", subcore_axis_name="s",
      num_cores=NC, num_subcores=NS)

  buf_set = lambda: [
      pltpu.VMEM((BLK,), i32),
      pltpu.VMEM((BLK,), i32),
      pltpu.VMEM((BLK, F), f32),
      pltpu.VMEM((BLK, F), f32),
      pltpu.VMEM((BLK,), i32),
      pltpu.VMEM((BLK, F), f32),
      pltpu.VMEM((BLK, L), f32),
      pltpu.SemaphoreType.DMA,
      pltpu.SemaphoreType.DMA,
  ]

  @functools.partial(
      pl.kernel,
      out_type=(jax.ShapeDtypeStruct((NC, NR, F), f32),
                jax.ShapeDtypeStruct((NC, NR, L), f32)),
      mesh=mesh,
      compiler_params=pltpu.CompilerParams(use_tc_tiling_on_sc=False,
                                           needs_layout_passes=False),
      scratch_types=buf_set() + buf_set() + [
          pltpu.VMEM((F,), f32),
          pltpu.VMEM((2 * L,), f32),
          pltpu.VMEM((2 * L,), f32),
          pltpu.VMEM((2 * L,), f32),
          pltpu.VMEM((2 * L,), f32),
          pltpu.VMEM((ZR, F), f32),
          pltpu.VMEM((ZR, L), f32),
          pltpu.VMEM_SHARED((NR, F), f32),
          pltpu.VMEM_SHARED((NR, L), f32),
      ],
  )
  def k(tbl_l, tbl_r, srcp, dstp, attp, outm, outs,
        srcA, dstA, xjA, xiA, scatA, msgA, alpA, gsemA, ssemA,
        srcB, dstB, xjB, xiB, scatB, msgB, alpB, gsemB, ssemB,
        att_v, sw0, sw1, sw2, sw3,
        zm_v, zs_v, accm_sh, accs_sh):
    cid = lax.axis_index("c")
    sid = lax.axis_index("s")
    wid = cid * NS + sid
    sws = [sw0, sw1, sw2, sw3]
    bufA = (srcA, dstA, xjA, xiA, scatA, msgA, alpA, gsemA, ssemA)
    bufB = (srcB, dstB, xjB, xiB, scatB, msgB, alpB, gsemB, ssemB)

    zero16 = jnp.zeros((L,), f32)

    def zrow(r, carry):
      for v in range(NV):
        zm_v[r, pl.ds(v * L, L)] = zero16
      zs_v[r, pl.ds(0, L)] = zero16
      return carry
    _fori(ZR, zrow)

    def zcopy(kk, carry):
      off = sid * STRIDE + kk * ZR
      pltpu.sync_copy(zm_v, accm_sh.at[pl.ds(off, ZR)])
      pltpu.sync_copy(zs_v, accs_sh.at[pl.ds(off, ZR)])
      return carry
    _fori(STRIDE // ZR, zcopy)
    plsc.subcore_barrier()

    pltpu.sync_copy(attp, att_v)
    atts = [att_v[pl.ds(v * L, L)] for v in range(NV)]

    tile_base = wid * per_tile

    def start_block(b, buf):
      src_v, dst_v, xj_v, xi_v = buf[0], buf[1], buf[2], buf[3]
      gsem = buf[7]
      base = tile_base + b * BLK
      pltpu.sync_copy(srcp.at[pl.ds(base, BLK)], src_v)
      pltpu.sync_copy(dstp.at[pl.ds(base, BLK)], dst_v)
      pltpu.async_copy(tbl_l.at[src_v], xj_v, gsem)
      pltpu.async_copy(tbl_r.at[dst_v], xi_v, gsem)

    def drain_scatter(buf):
      scat_v, msg_v, alp_v, ssem = buf[4], buf[5], buf[6], buf[8]
      pltpu.make_async_copy(msg_v, accm_sh.at[scat_v], ssem).wait()
      pltpu.make_async_copy(alp_v, accs_sh.at[scat_v], ssem).wait()

    def process_block(b, cur, nxt):
      src_v, dst_v, xj_v, xi_v, scat_v, msg_v, alp_v, gsem, ssem = cur

      @pl.when(b + 1 < nblk)
      def _():
        start_block(b + 1, nxt)

      @pl.when(b >= 2)
      def _():
        drain_scatter(cur)

      base = tile_base + b * BLK
      for g in range(BLK // L):
        sv = src_v[pl.ds(g * L, L)]
        dv = dst_v[pl.ds(g * L, L)]
        eid = base + g * L + lax.iota(i32, L)
        valid = jnp.where(eid < E, sv != dv, eid < Et)
        scat_v[pl.ds(g * L, L)] = jnp.where(valid, dv, TRASH)

      pltpu.make_async_copy(tbl_l.at[pl.ds(0, BLK)], xj_v, gsem).wait()
      pltpu.make_async_copy(tbl_l.at[pl.ds(0, BLK)], xi_v, gsem).wait()

      def edge(ii, ecarry):
        e0 = ii * U
        for u in range(U):
          e = e0 + u
          xjs = [xj_v[e, pl.ds(v * L, L)] for v in range(NV)]
          xis = [xi_v[e, pl.ds(v * L, L)] for v in range(NV)]
          msgs, a = compute_fn(xjs, xis, atts, sws[u])
          for v in range(NV):
            msg_v[e, pl.ds(v * L, L)] = msgs[v]
          alp_v[e, pl.ds(0, L)] = a
        return ecarry
      _fori(BLK // U, edge)

      pltpu.async_copy(msg_v, accm_sh.at[scat_v], ssem, add=True)
      pltpu.async_copy(alp_v, accs_sh.at[scat_v], ssem, add=True)

    start_block(jnp.int32(0), bufA)

    def pair_body(i, carry):
      b = i * 2
      process_block(b, bufA, bufB)
      process_block(b + 1, bufB, bufA)
      return carry
    _fori(nblk // 2, pair_body)

    drain_scatter(bufA)
    drain_scatter(bufB)

    plsc.subcore_barrier()
    off = sid * STRIDE
    pltpu.sync_copy(accm_sh.at[pl.ds(off, STRIDE)],
                    outm.at[cid, pl.ds(off, STRIDE)])
    pltpu.sync_copy(accs_sh.at[pl.ds(off, STRIDE)],
                    outs.at[cid, pl.ds(off, STRIDE)])

  return k


def _tc_linear2(x, wl, wr):
  n = x.shape[0]
  m = wl.shape[1]

  def body(x_ref, wl_ref, wr_ref, ol_ref, or_ref):
    xv = x_ref[...]
    ol_ref[...] = jnp.dot(xv, wl_ref[...], preferred_element_type=f32,
                         precision=lax.Precision.HIGHEST)
    or_ref[...] = jnp.dot(xv, wr_ref[...], preferred_element_type=f32,
                         precision=lax.Precision.HIGHEST)

  return pl.pallas_call(
      body,
      out_shape=(jax.ShapeDtypeStruct((n, m), f32),
                 jax.ShapeDtypeStruct((n, m), f32)),
  )(x, wl, wr)


def _tc_combine1(pm, ps, b1p, wl2, wr2, ch):
  NR = pm.shape[1]
  F = pm.shape[2]
  m = wl2.shape[1]

  def body(pm_ref, ps_ref, b_ref, wl_ref, wr_ref, ol_ref, or_ref):
    accm = pm_ref[0] + pm_ref[1]
    accs = ps_ref[0] + ps_ref[1]
    denom = jnp.tile(accs, (1, F // L)) + 1e-16
    h = accm / denom + b_ref[...]
    h = jnp.where(h > 0, h, jnp.exp(h) - 1.0)
    ol_ref[...] = jnp.dot(h, wl_ref[...], preferred_element_type=f32,
                          precision=lax.Precision.HIGHEST)
    or_ref[...] = jnp.dot(h, wr_ref[...], preferred_element_type=f32,
                          precision=lax.Precision.HIGHEST)

  return pl.pallas_call(
      body,
      out_shape=(jax.ShapeDtypeStruct((NR, m), f32),
                 jax.ShapeDtypeStruct((NR, m), f32)),
  )(pm, ps, b1p, wl2, wr2)


def _tc_finalize(pm, ps, b2, n, dout):
  def body(pm_ref, ps_ref, b_ref, h2_ref, out_ref, pred_ref):
    accm = pm_ref[0] + pm_ref[1]
    accs = ps_ref[0] + ps_ref[1]
    h2full = accm / (accs + 1e-16) + b_ref[...]
    h2 = h2full[:n]
    mx = jnp.max(h2, axis=1, keepdims=True)
    lse = jnp.log(jnp.sum(jnp.exp(h2 - mx), axis=1, keepdims=True)) + mx
    o = h2 - lse
    h2_ref[...] = h2
    out_ref[...] = o
    eq = o >= jnp.max(o, axis=1, keepdims=True)
    idx = lax.broadcasted_iota(i32, o.shape, 1)
    pred_ref[...] = jnp.min(jnp.where(eq, idx, dout), axis=1, keepdims=True)

  return pl.pallas_call(
      body,
      out_shape=(jax.ShapeDtypeStruct((n, dout), f32),
                 jax.ShapeDtypeStruct((n, dout), f32),
                 jax.ShapeDtypeStruct((n, 1), i32)),
  )(pm, ps, b2)


def kernel(x, edge_index, Wl1, Wr1, att1, b1, Wl2, Wr2, att2, b2):
  N, DIN = x.shape
  E = edge_index.shape[1]
  H, C = att1.shape
  HC = H * C
  DOUT = Wl2.shape[1]

  NR = -(-N // (NS * ZR)) * (NS * ZR)

  x32 = x.astype(f32)
  Wl1p = Wl1.astype(f32).reshape(DIN, H, C).transpose(0, 2, 1).reshape(DIN, HC)
  Wr1p = Wr1.astype(f32).reshape(DIN, H, C).transpose(0, 2, 1).reshape(DIN, HC)
  att1p = att1.astype(f32).T.reshape(HC)
  b1p = b1.astype(f32).reshape(H, C).T.reshape(1, HC)
  Wl2p = Wl2.astype(f32).reshape(H, C, DOUT).transpose(1, 0, 2).reshape(HC, DOUT)
  Wr2p = Wr2.astype(f32).reshape(H, C, DOUT).transpose(1, 0, 2).reshape(HC, DOUT)
  att2p = att2.astype(f32).reshape(DOUT)
  b2p = b2.astype(f32).reshape(1, DOUT)

  src = edge_index[0].astype(i32)
  dst = edge_index[1].astype(i32)
  loop = jnp.arange(N, dtype=i32)
  Et = E + N
  nblk = 2 * (-(-Et // (NW * BLK * 2)))
  Etp = nblk * BLK * NW
  pad = Etp - Et
  srcp = jnp.concatenate([src, loop, jnp.zeros((pad,), i32)])
  dstp = jnp.concatenate([dst, loop, jnp.zeros((pad,), i32)])

  xl1, xr1 = _tc_linear2(x32, Wl1p, Wr1p)
  edge1 = _make_edge_kernel(N, HC, nblk, E, Et, _compute_l1, NR)
  pm1, ps1 = edge1(xl1, xr1, srcp, dstp, att1p)

  xl2, xr2 = _tc_combine1(pm1, ps1, b1p, Wl2p, Wr2p, C)

  edge2 = _make_edge_kernel(NR, DOUT, nblk, E, Et, _compute_l2, NR)
  pm2, ps2 = edge2(xl2, xr2, srcp, dstp, att2p)

  h2, out, preds = _tc_finalize(pm2, ps2, b2p, N, DOUT)
  return (h2.astype(jnp.float64), out.astype(jnp.float64),
          preds.reshape(N).astype(jnp.int64))
